# baseline (speedup 1.0000x reference)
import functools

import jax
import jax.numpy as jnp
from jax import lax
from jax.experimental import pallas as pl
from jax.experimental.pallas import tpu as pltpu

N_EXPERTS = 8
TOP_K = 2
ALPHA = 0.001
TOKEN_BLOCK = 1024


def _gate_block(x_ref, w_ref, idx_ref, wgt_ref, aux_ref, acc_ref):
    i = pl.program_id(0)
    nb = pl.num_programs(0)

    x = x_ref[...]
    w = w_ref[...]
    logits = lax.dot_general(
        x, w, (((1,), (1,)), ((), ())), preferred_element_type=jnp.float32
    )

    m = jnp.max(logits, axis=1, keepdims=True)
    e = jnp.exp(logits - m)
    p = e / jnp.sum(e, axis=1, keepdims=True)

    iota = lax.broadcasted_iota(jnp.int32, p.shape, 1)
    m1 = jnp.max(p, axis=1, keepdims=True)
    idx1 = jnp.min(jnp.where(p == m1, iota, N_EXPERTS), axis=1, keepdims=True)
    is1 = iota == idx1
    p2 = jnp.where(is1, -1.0, p)
    m2 = jnp.max(p2, axis=1, keepdims=True)
    idx2 = jnp.min(jnp.where(p2 == m2, iota, N_EXPERTS), axis=1, keepdims=True)
    is2 = iota == idx2

    denom = m1 + m2 + 1e-20
    idx_ref[0, :, :] = jnp.concatenate([idx1, idx2], axis=1)
    wgt_ref[0, :, :] = jnp.concatenate([m1 / denom, m2 / denom], axis=1)

    pi_part = jnp.sum(p, axis=0, keepdims=True)
    cnt_part = jnp.sum(
        jnp.where(is1, 1.0, 0.0) + jnp.where(is2, 1.0, 0.0), axis=0, keepdims=True
    )
    part = jnp.concatenate([pi_part, cnt_part], axis=0)

    @pl.when(i == 0)
    def _init():
        acc_ref[...] = part

    @pl.when(i > 0)
    def _acc():
        acc_ref[...] += part

    @pl.when(i == nb - 1)
    def _fin():
        acc = acc_ref[...]
        total = nb * x.shape[0]
        pi = acc[0, :] / total
        fi = acc[1, :] * (N_EXPERTS / (total * TOP_K))
        aux = jnp.sum(pi * fi) * ALPHA
        aux_ref[...] = jnp.full((8, 128), aux, jnp.float32)


def kernel(hidden_states, kernel):
    bsz, seq_len, h = hidden_states.shape
    t = bsz * seq_len
    hs = hidden_states.reshape(t, h)
    tb = TOKEN_BLOCK
    nb = t // tb

    idx3, wgt3, aux = pl.pallas_call(
        _gate_block,
        grid=(nb,),
        in_specs=[
            pl.BlockSpec((tb, h), lambda i: (i, 0)),
            pl.BlockSpec((N_EXPERTS, h), lambda i: (0, 0)),
        ],
        out_specs=[
            pl.BlockSpec((1, tb, TOP_K), lambda i: (i, 0, 0)),
            pl.BlockSpec((1, tb, TOP_K), lambda i: (i, 0, 0)),
            pl.BlockSpec((8, 128), lambda i: (0, 0)),
        ],
        out_shape=[
            jax.ShapeDtypeStruct((nb, tb, TOP_K), jnp.int32),
            jax.ShapeDtypeStruct((nb, tb, TOP_K), jnp.float32),
            jax.ShapeDtypeStruct((8, 128), jnp.float32),
        ],
        scratch_shapes=[pltpu.VMEM((2, N_EXPERTS), jnp.float32)],
        compiler_params=pltpu.CompilerParams(
            dimension_semantics=("arbitrary",),
        ),
    )(hs, kernel)

    topk_idx = idx3.reshape(t, TOP_K)
    topk_weight = wgt3.reshape(t, TOP_K)
    aux_loss = aux[0, 0]
    return (topk_idx, topk_weight, aux_loss)


# speedup vs baseline: 1.5056x; 1.5056x over previous
import jax
import jax.numpy as jnp
from jax import lax
from jax.experimental import pallas as pl
from jax.experimental.pallas import tpu as pltpu

N_EXPERTS = 8
TOP_K = 2
ALPHA = 0.001
TOKEN_BLOCK = 1024


def _gate_block(x_ref, w_ref, idx_ref, wgt_ref, aux_ref, acc_ref):
    i = pl.program_id(0)
    nb = pl.num_programs(0)

    x = x_ref[...]
    w = w_ref[...]
    logits = lax.dot_general(
        w, x, (((1,), (1,)), ((), ())), preferred_element_type=jnp.float32
    )

    m = jnp.max(logits, axis=0, keepdims=True)
    e = jnp.exp(logits - m)
    p = e / jnp.sum(e, axis=0, keepdims=True)

    iota = lax.broadcasted_iota(jnp.int32, p.shape, 0)
    m1 = jnp.max(p, axis=0, keepdims=True)
    idx1 = jnp.min(jnp.where(p == m1, iota, N_EXPERTS), axis=0, keepdims=True)
    is1 = iota == idx1
    p2 = jnp.where(is1, -1.0, p)
    m2 = jnp.max(p2, axis=0, keepdims=True)
    idx2 = jnp.min(jnp.where(p2 == m2, iota, N_EXPERTS), axis=0, keepdims=True)
    is2 = iota == idx2

    denom = m1 + m2 + 1e-20
    idx_ref[0, :, :] = jnp.concatenate([idx1, idx2], axis=0)
    wgt_ref[0, :, :] = jnp.concatenate([m1 / denom, m2 / denom], axis=0)

    part = jnp.concatenate(
        [p, jnp.where(is1 | is2, 1.0, 0.0)], axis=0
    )

    @pl.when(i == 0)
    def _init():
        acc_ref[...] = part

    @pl.when(i > 0)
    def _acc():
        acc_ref[...] += part

    @pl.when(i == nb - 1)
    def _fin():
        acc = jnp.sum(acc_ref[...], axis=1)
        total = nb * x.shape[0]
        pi = acc[:N_EXPERTS] / total
        fi = acc[N_EXPERTS:] * (N_EXPERTS / (total * TOP_K))
        aux = jnp.sum(pi * fi) * ALPHA
        aux_ref[...] = jnp.full((8, 128), aux, jnp.float32)


def kernel(hidden_states, kernel):
    bsz, seq_len, h = hidden_states.shape
    t = bsz * seq_len
    hs = hidden_states.reshape(t, h)
    tb = TOKEN_BLOCK
    nb = t // tb

    idx3, wgt3, aux = pl.pallas_call(
        _gate_block,
        grid=(nb,),
        in_specs=[
            pl.BlockSpec((tb, h), lambda i: (i, 0)),
            pl.BlockSpec((N_EXPERTS, h), lambda i: (0, 0)),
        ],
        out_specs=[
            pl.BlockSpec((1, TOP_K, tb), lambda i: (i, 0, 0)),
            pl.BlockSpec((1, TOP_K, tb), lambda i: (i, 0, 0)),
            pl.BlockSpec((8, 128), lambda i: (0, 0)),
        ],
        out_shape=[
            jax.ShapeDtypeStruct((nb, TOP_K, tb), jnp.int32),
            jax.ShapeDtypeStruct((nb, TOP_K, tb), jnp.float32),
            jax.ShapeDtypeStruct((8, 128), jnp.float32),
        ],
        scratch_shapes=[pltpu.VMEM((2 * N_EXPERTS, tb), jnp.float32)],
        compiler_params=pltpu.CompilerParams(
            dimension_semantics=("arbitrary",),
        ),
    )(hs, kernel)

    topk_idx = idx3.transpose(0, 2, 1).reshape(t, TOP_K)
    topk_weight = wgt3.transpose(0, 2, 1).reshape(t, TOP_K)
    aux_loss = aux[0, 0]
    return (topk_idx, topk_weight, aux_loss)
